# Initial kernel scaffold; baseline (speedup 1.0000x reference)
import jax
import jax.numpy as jnp
from jax.experimental import pallas as pl


def kernel(dense_x, lS_i, emb_tables, bW0, bb0, bW1, bb1, bW2, bb2, tW0, tb0, tW1, tb1, tW2, tb2):
    raise NotImplementedError("write your pallas kernel here")


# speedup vs baseline: 6.8013x; 6.8013x over previous
import functools
import numpy as np
import jax
import jax.numpy as jnp
from jax import lax
from jax.experimental import pallas as pl
from jax.experimental.pallas import tpu as pltpu
from jax.experimental.pallas import tpu_sc as plsc

B = 16384
F = 26
V = 100000
D = 16

NC, NS, L = 2, 16, 16
NW = NC * NS
ITEMS = F * B
PER_W = ITEMS // NW
CHUNK = 3328
NCHUNK = PER_W // CHUNK

_mesh = plsc.VectorSubcoreMesh(core_axis_name="c", subcore_axis_name="s")


@functools.partial(
    pl.kernel,
    out_type=jax.ShapeDtypeStruct((ITEMS, D), jnp.float32),
    mesh=_mesh,
    scratch_types=[
        pltpu.VMEM((CHUNK,), jnp.int32),
        pltpu.VMEM((CHUNK, D), jnp.float32),
        pltpu.SemaphoreType.DMA,
    ],
    compiler_params=pltpu.CompilerParams(use_tc_tiling_on_sc=False),
)
def _sc_gather(tables_hbm, idx_hbm, out_hbm, idx_v, rows_v, sem):
    wid = lax.axis_index("s") * NC + lax.axis_index("c")
    base = wid * PER_W
    lane = lax.iota(jnp.int32, L)
    for c in range(NCHUNK):
        cbase = base + c * CHUNK
        pltpu.sync_copy(idx_hbm.at[pl.ds(cbase, CHUNK)], idx_v)

        def _off(j, _):
            pos = cbase + j * L + lane
            f = lax.shift_right_logical(pos, 14)
            idx_v[pl.ds(j * L, L)] = idx_v[pl.ds(j * L, L)] + f * V
            return _

        lax.fori_loop(0, CHUNK // L, _off, None)

        pltpu.async_copy(tables_hbm.at[idx_v], rows_v, sem).wait()
        pltpu.sync_copy(rows_v, out_hbm.at[pl.ds(cbase, CHUNK)])


BB = 512
NBLK = B // BB
_LI, _LJ = np.tril_indices(F + 1, -1)


def _tc_body(xdT_ref, lyT_ref, bW0_ref, bb0_ref, bW1_ref, bb1_ref,
             bW2_ref, bb2_ref, tW0_ref, tb0_ref, tW1_ref, tb1_ref,
             tW2_ref, tb2_ref, out_ref, rt_ref):
    f32 = jnp.float32
    x = xdT_ref[...]
    h = jnp.maximum(jnp.dot(bW0_ref[...], x, preferred_element_type=f32)
                    + bb0_ref[...], 0.0)
    h = jnp.maximum(jnp.dot(bW1_ref[...], h, preferred_element_type=f32)
                    + bb1_ref[...], 0.0)
    x3 = jnp.maximum(jnp.dot(bW2_ref[...], h, preferred_element_type=f32)
                     + bb2_ref[...], 0.0)

    ly = lyT_ref[...]
    ts = [x3] + [lax.slice(ly, (f * D, 0), ((f + 1) * D, BB))
                 for f in range(F)]

    rt_ref[0:D, :] = x3
    rt_ref[pl.ds(D + len(_LI), 1), :] = jnp.zeros((1, BB), f32)
    for p in range(len(_LI)):
        z = jnp.sum(ts[_LI[p]] * ts[_LJ[p]], axis=0, keepdims=True)
        rt_ref[pl.ds(D + p, 1), :] = z

    r = rt_ref[...]
    h = jnp.maximum(jnp.dot(tW0_ref[...], r, preferred_element_type=f32)
                    + tb0_ref[...], 0.0)
    h = jnp.maximum(jnp.dot(tW1_ref[...], h, preferred_element_type=f32)
                    + tb1_ref[...], 0.0)
    o = jnp.dot(tW2_ref[...], h, preferred_element_type=f32) + tb2_ref[...]
    out_ref[...] = 1.0 / (1.0 + jnp.exp(-o))


def _full(shape):
    return pl.BlockSpec(shape, lambda i: (0,) * len(shape))


_tc_call = pl.pallas_call(
    _tc_body,
    grid=(NBLK,),
    in_specs=[
        pl.BlockSpec((13, BB), lambda i: (0, i)),
        pl.BlockSpec((F * D, BB), lambda i: (0, i)),
        _full((512, 13)), _full((512, 1)),
        _full((256, 512)), _full((256, 1)),
        _full((16, 256)), _full((16, 1)),
        _full((512, 368)), _full((512, 1)),
        _full((256, 512)), _full((256, 1)),
        _full((1, 256)), _full((1, 1)),
    ],
    out_specs=pl.BlockSpec((1, BB), lambda i: (0, i)),
    out_shape=jax.ShapeDtypeStruct((1, B), jnp.float32),
    scratch_shapes=[pltpu.VMEM((368, BB), jnp.float32)],
)


def kernel(dense_x, lS_i, emb_tables, bW0, bb0, bW1, bb1, bW2, bb2,
           tW0, tb0, tW1, tb1, tW2, tb2):
    idx_flat = lS_i.astype(jnp.int32).reshape(ITEMS)
    tables_flat = emb_tables.reshape(F * V, D)
    ly_flat = _sc_gather(tables_flat, idx_flat)
    lyT = ly_flat.reshape(F, B, D).transpose(0, 2, 1).reshape(F * D, B)

    xdT = dense_x.T
    tW0p = jnp.pad(tW0, ((0, 0), (0, 1)))
    pT = _tc_call(xdT, lyT,
                  bW0, bb0[:, None], bW1, bb1[:, None], bW2, bb2[:, None],
                  tW0p, tb0[:, None], tW1, tb1[:, None],
                  tW2, tb2[:, None])
    return pT.reshape(B, 1)


# speedup vs baseline: 28.3465x; 4.1678x over previous
import functools
import numpy as np
import jax
import jax.numpy as jnp
from jax import lax
from jax.experimental import pallas as pl
from jax.experimental.pallas import tpu as pltpu
from jax.experimental.pallas import tpu_sc as plsc

B = 16384
F = 26
V = 100000
D = 16

NC, NS, L = 2, 16, 16
NW = NC * NS
ROWS = F * D
RPT = ROWS // NW
OCHUNK = 8192

_mesh = plsc.VectorSubcoreMesh(core_axis_name="c", subcore_axis_name="s")


@functools.partial(
    pl.kernel,
    out_type=jax.ShapeDtypeStruct((ROWS, B), jnp.float32),
    mesh=_mesh,
    scratch_types=[
        pltpu.VMEM((V,), jnp.float32),
        pltpu.VMEM((B,), jnp.int32),
        pltpu.VMEM((OCHUNK,), jnp.float32),
    ],
    compiler_params=pltpu.CompilerParams(needs_layout_passes=False),
)
def _sc_gather(tabT_hbm, idx_hbm, out_hbm, row_v, idx_v, out_v):
    wid = lax.axis_index("s") * NC + lax.axis_index("c")
    for i in range(RPT):
        r = wid * RPT + i
        f = r // D
        d = r % D
        pltpu.sync_copy(tabT_hbm.at[f, d, :], row_v)
        pltpu.sync_copy(idx_hbm.at[f, :], idx_v)
        for h in range(B // OCHUNK):

            def _gat(j, _, h=h):
                iv = idx_v[pl.ds(h * OCHUNK + j * L, L)]
                out_v[pl.ds(j * L, L)] = plsc.load_gather(row_v, [iv])
                return _

            lax.fori_loop(0, OCHUNK // L, _gat, None)
            pltpu.sync_copy(out_v, out_hbm.at[r, pl.ds(h * OCHUNK, OCHUNK)])


BB = 512
NBLK = B // BB
_LI, _LJ = np.tril_indices(F + 1, -1)


def _tc_body(xdT_ref, lyT_ref, bW0_ref, bb0_ref, bW1_ref, bb1_ref,
             bW2_ref, bb2_ref, tW0_ref, tb0_ref, tW1_ref, tb1_ref,
             tW2_ref, tb2_ref, out_ref, rt_ref):
    f32 = jnp.float32
    x = xdT_ref[...]
    h = jnp.maximum(jnp.dot(bW0_ref[...], x, preferred_element_type=f32)
                    + bb0_ref[...], 0.0)
    h = jnp.maximum(jnp.dot(bW1_ref[...], h, preferred_element_type=f32)
                    + bb1_ref[...], 0.0)
    x3 = jnp.maximum(jnp.dot(bW2_ref[...], h, preferred_element_type=f32)
                     + bb2_ref[...], 0.0)

    ly = lyT_ref[...]
    ts = [x3] + [lax.slice(ly, (f * D, 0), ((f + 1) * D, BB))
                 for f in range(F)]

    rt_ref[0:D, :] = x3
    rt_ref[pl.ds(D + len(_LI), 1), :] = jnp.zeros((1, BB), f32)
    for p in range(len(_LI)):
        z = jnp.sum(ts[_LI[p]] * ts[_LJ[p]], axis=0, keepdims=True)
        rt_ref[pl.ds(D + p, 1), :] = z

    r = rt_ref[...]
    h = jnp.maximum(jnp.dot(tW0_ref[...], r, preferred_element_type=f32)
                    + tb0_ref[...], 0.0)
    h = jnp.maximum(jnp.dot(tW1_ref[...], h, preferred_element_type=f32)
                    + tb1_ref[...], 0.0)
    o = jnp.dot(tW2_ref[...], h, preferred_element_type=f32) + tb2_ref[...]
    out_ref[...] = 1.0 / (1.0 + jnp.exp(-o))


def _full(shape):
    return pl.BlockSpec(shape, lambda i: (0,) * len(shape))


_tc_call = pl.pallas_call(
    _tc_body,
    grid=(NBLK,),
    in_specs=[
        pl.BlockSpec((13, BB), lambda i: (0, i)),
        pl.BlockSpec((F * D, BB), lambda i: (0, i)),
        _full((512, 13)), _full((512, 1)),
        _full((256, 512)), _full((256, 1)),
        _full((16, 256)), _full((16, 1)),
        _full((512, 368)), _full((512, 1)),
        _full((256, 512)), _full((256, 1)),
        _full((1, 256)), _full((1, 1)),
    ],
    out_specs=pl.BlockSpec((1, BB), lambda i: (0, i)),
    out_shape=jax.ShapeDtypeStruct((1, B), jnp.float32),
    scratch_shapes=[pltpu.VMEM((368, BB), jnp.float32)],
)


def kernel(dense_x, lS_i, emb_tables, bW0, bb0, bW1, bb1, bW2, bb2,
           tW0, tb0, tW1, tb1, tW2, tb2):
    tabT = emb_tables.transpose(0, 2, 1)
    lyT = _sc_gather(tabT, lS_i.astype(jnp.int32))

    xdT = dense_x.T
    tW0p = jnp.pad(tW0, ((0, 0), (0, 1)))
    pT = _tc_call(xdT, lyT,
                  bW0, bb0[:, None], bW1, bb1[:, None], bW2, bb2[:, None],
                  tW0p, tb0[:, None], tW1, tb1[:, None],
                  tW2, tb2[:, None])
    return pT.reshape(B, 1)


# speedup vs baseline: 39.4611x; 1.3921x over previous
import functools
import numpy as np
import jax
import jax.numpy as jnp
from jax import lax
from jax.experimental import pallas as pl
from jax.experimental.pallas import tpu as pltpu
from jax.experimental.pallas import tpu_sc as plsc

B = 16384
F = 26
V = 100000
D = 16

NC, NS, L = 2, 16, 16
NW = NC * NS
ROWS = F * D
RPT = ROWS // NW
OCHUNK = 4096

_mesh = plsc.VectorSubcoreMesh(core_axis_name="c", subcore_axis_name="s")


@functools.partial(
    pl.kernel,
    out_type=jax.ShapeDtypeStruct((ROWS, B), jnp.float32),
    mesh=_mesh,
    scratch_types=[
        pltpu.VMEM((V,), jnp.float32),
        pltpu.VMEM((B,), jnp.int32),
        pltpu.VMEM((OCHUNK,), jnp.float32),
        pltpu.VMEM((OCHUNK,), jnp.float32),
        pltpu.SemaphoreType.DMA,
        pltpu.SemaphoreType.DMA,
        pltpu.SemaphoreType.DMA,
        pltpu.SemaphoreType.DMA,
    ],
    compiler_params=pltpu.CompilerParams(needs_layout_passes=False),
)
def _sc_gather(tabT_hbm, idx_hbm, out_hbm, row_v, idx_v, out_v0, out_v1,
               rsem, isem, osem0, osem1):
    wid = lax.axis_index("s") * NC + lax.axis_index("c")
    osems = (osem0, osem1)
    obufs = (out_v0, out_v1)
    pending = [None, None]
    for i in range(RPT):
        r = wid * RPT + i
        f = r // D
        d = r % D
        crow = pltpu.async_copy(tabT_hbm.at[f, d, :], row_v, rsem)
        cidx = pltpu.async_copy(idx_hbm.at[f, :], idx_v, isem)
        crow.wait()
        cidx.wait()
        for h in range(B // OCHUNK):
            ob = obufs[h % 2]
            if pending[h % 2] is not None:
                pending[h % 2].wait()

            @plsc.parallel_loop(0, OCHUNK // L, unroll=4)
            def _gat(j, h=h, ob=ob):
                iv = idx_v[pl.ds(h * OCHUNK + j * L, L)]
                ob[pl.ds(j * L, L)] = plsc.load_gather(row_v, [iv])

            pending[h % 2] = pltpu.async_copy(
                ob, out_hbm.at[r, pl.ds(h * OCHUNK, OCHUNK)], osems[h % 2])
    for h in range(2):
        if pending[h] is not None:
            pending[h].wait()


BB = 512
NBLK = B // BB
_LI, _LJ = np.tril_indices(F + 1, -1)


def _tc_body(xdT_ref, lyT_ref, bW0_ref, bb0_ref, bW1_ref, bb1_ref,
             bW2_ref, bb2_ref, tW0_ref, tb0_ref, tW1_ref, tb1_ref,
             tW2_ref, tb2_ref, out_ref, rt_ref):
    f32 = jnp.float32
    x = xdT_ref[...]
    h = jnp.maximum(jnp.dot(bW0_ref[...], x, preferred_element_type=f32)
                    + bb0_ref[...], 0.0)
    h = jnp.maximum(jnp.dot(bW1_ref[...], h, preferred_element_type=f32)
                    + bb1_ref[...], 0.0)
    x3 = jnp.maximum(jnp.dot(bW2_ref[...], h, preferred_element_type=f32)
                     + bb2_ref[...], 0.0)

    ly = lyT_ref[...]
    ts = [x3] + [lax.slice(ly, (f * D, 0), ((f + 1) * D, BB))
                 for f in range(F)]

    rt_ref[0:D, :] = x3
    rt_ref[pl.ds(D + len(_LI), 1), :] = jnp.zeros((1, BB), f32)
    for p in range(len(_LI)):
        z = jnp.sum(ts[_LI[p]] * ts[_LJ[p]], axis=0, keepdims=True)
        rt_ref[pl.ds(D + p, 1), :] = z

    r = rt_ref[...]
    h = jnp.maximum(jnp.dot(tW0_ref[...], r, preferred_element_type=f32)
                    + tb0_ref[...], 0.0)
    h = jnp.maximum(jnp.dot(tW1_ref[...], h, preferred_element_type=f32)
                    + tb1_ref[...], 0.0)
    o = jnp.dot(tW2_ref[...], h, preferred_element_type=f32) + tb2_ref[...]
    out_ref[...] = 1.0 / (1.0 + jnp.exp(-o))


def _full(shape):
    return pl.BlockSpec(shape, lambda i: (0,) * len(shape))


_tc_call = pl.pallas_call(
    _tc_body,
    grid=(NBLK,),
    in_specs=[
        pl.BlockSpec((13, BB), lambda i: (0, i)),
        pl.BlockSpec((F * D, BB), lambda i: (0, i)),
        _full((512, 13)), _full((512, 1)),
        _full((256, 512)), _full((256, 1)),
        _full((16, 256)), _full((16, 1)),
        _full((512, 368)), _full((512, 1)),
        _full((256, 512)), _full((256, 1)),
        _full((1, 256)), _full((1, 1)),
    ],
    out_specs=pl.BlockSpec((1, BB), lambda i: (0, i)),
    out_shape=jax.ShapeDtypeStruct((1, B), jnp.float32),
    scratch_shapes=[pltpu.VMEM((368, BB), jnp.float32)],
)


def kernel(dense_x, lS_i, emb_tables, bW0, bb0, bW1, bb1, bW2, bb2,
           tW0, tb0, tW1, tb1, tW2, tb2):
    tabT = emb_tables.transpose(0, 2, 1)
    lyT = _sc_gather(tabT, lS_i.astype(jnp.int32))

    xdT = dense_x.T
    tW0p = jnp.pad(tW0, ((0, 0), (0, 1)))
    pT = _tc_call(xdT, lyT,
                  bW0, bb0[:, None], bW1, bb1[:, None], bW2, bb2[:, None],
                  tW0p, tb0[:, None], tW1, tb1[:, None],
                  tW2, tb2[:, None])
    return pT.reshape(B, 1)


# speedup vs baseline: 46.3172x; 1.1737x over previous
import functools
import numpy as np
import jax
import jax.numpy as jnp
from jax import lax
from jax.experimental import pallas as pl
from jax.experimental.pallas import tpu as pltpu
from jax.experimental.pallas import tpu_sc as plsc

B = 16384
F = 26
V = 100000
D = 16

NC, NS, L = 2, 16, 16
NW = NC * NS
ROWS = F * D
RPT = ROWS // NW
OCHUNK = 4096

_mesh = plsc.VectorSubcoreMesh(core_axis_name="c", subcore_axis_name="s")


@functools.partial(
    pl.kernel,
    out_type=jax.ShapeDtypeStruct((ROWS, B), jnp.float32),
    mesh=_mesh,
    scratch_types=[
        pltpu.VMEM((V,), jnp.float32),
        pltpu.VMEM((B,), jnp.int32),
        pltpu.VMEM((OCHUNK,), jnp.float32),
        pltpu.VMEM((OCHUNK,), jnp.float32),
        pltpu.SemaphoreType.DMA,
        pltpu.SemaphoreType.DMA,
        pltpu.SemaphoreType.DMA,
        pltpu.SemaphoreType.DMA,
    ],
    compiler_params=pltpu.CompilerParams(needs_layout_passes=False),
)
def _sc_gather(tabT_hbm, idx_hbm, out_hbm, row_v, idx_v, out_v0, out_v1,
               rsem, isem, osem0, osem1):
    wid = lax.axis_index("s") * NC + lax.axis_index("c")
    osems = (osem0, osem1)
    obufs = (out_v0, out_v1)
    pending = [None, None]
    for i in range(RPT):
        r = wid * RPT + i
        f = r // D
        d = r % D
        crow = pltpu.async_copy(tabT_hbm.at[f, d, :], row_v, rsem)
        cidx = pltpu.async_copy(idx_hbm.at[f, :], idx_v, isem)
        crow.wait()
        cidx.wait()
        for h in range(B // OCHUNK):
            ob = obufs[h % 2]
            if pending[h % 2] is not None:
                pending[h % 2].wait()

            @plsc.parallel_loop(0, OCHUNK // L, unroll=4)
            def _gat(j, h=h, ob=ob):
                iv = idx_v[pl.ds(h * OCHUNK + j * L, L)]
                ob[pl.ds(j * L, L)] = plsc.load_gather(row_v, [iv])

            pending[h % 2] = pltpu.async_copy(
                ob, out_hbm.at[r, pl.ds(h * OCHUNK, OCHUNK)], osems[h % 2])
    for h in range(2):
        if pending[h] is not None:
            pending[h].wait()


BB = 512
NBLK = B // BB
_LI, _LJ = np.tril_indices(F + 1, -1)


def _tc_body(xdT_ref, lyT_ref, s8_ref, bW0_ref, bb0_ref, bW1_ref, bb1_ref,
             bW2_ref, bb2_ref, tW0_ref, tb0_ref, tW1_ref, tb1_ref,
             tW2_ref, tb2_ref, out_ref, rt_ref):
    f32 = jnp.float32
    x = xdT_ref[...]
    h = jnp.maximum(jnp.dot(bW0_ref[...], x, preferred_element_type=f32)
                    + bb0_ref[...], 0.0)
    h = jnp.maximum(jnp.dot(bW1_ref[...], h, preferred_element_type=f32)
                    + bb1_ref[...], 0.0)
    x3 = jnp.maximum(jnp.dot(bW2_ref[...], h, preferred_element_type=f32)
                     + bb2_ref[...], 0.0)

    ly = lyT_ref[...]
    ts = [x3] + [lax.slice(ly, (f * D, 0), ((f + 1) * D, BB))
                 for f in range(F)]

    rt_ref[0:D, :] = x3
    s8 = s8_ref[...]
    npair = len(_LI)
    for g in range((npair + 7) // 8):
        prods = []
        for u in range(8):
            p = g * 8 + u
            i, k = (_LI[p], _LJ[p]) if p < npair else (0, 0)
            prods.append(ts[i] * ts[k])
        zg = jnp.dot(s8, jnp.concatenate(prods, axis=0),
                     preferred_element_type=f32)
        rt_ref[pl.ds(D + g * 8, 8), :] = zg

    r = rt_ref[...]
    h = jnp.maximum(jnp.dot(tW0_ref[...], r, preferred_element_type=f32)
                    + tb0_ref[...], 0.0)
    h = jnp.maximum(jnp.dot(tW1_ref[...], h, preferred_element_type=f32)
                    + tb1_ref[...], 0.0)
    o = jnp.dot(tW2_ref[...], h, preferred_element_type=f32) + tb2_ref[...]
    out_ref[...] = 1.0 / (1.0 + jnp.exp(-o))


def _full(shape):
    return pl.BlockSpec(shape, lambda i: (0,) * len(shape))


_tc_call = pl.pallas_call(
    _tc_body,
    grid=(NBLK,),
    in_specs=[
        pl.BlockSpec((13, BB), lambda i: (0, i)),
        pl.BlockSpec((F * D, BB), lambda i: (0, i)),
        _full((8, 8 * D)),
        _full((512, 13)), _full((512, 1)),
        _full((256, 512)), _full((256, 1)),
        _full((16, 256)), _full((16, 1)),
        _full((512, 368)), _full((512, 1)),
        _full((256, 512)), _full((256, 1)),
        _full((1, 256)), _full((1, 1)),
    ],
    out_specs=pl.BlockSpec((1, BB), lambda i: (0, i)),
    out_shape=jax.ShapeDtypeStruct((1, B), jnp.float32),
    scratch_shapes=[pltpu.VMEM((368, BB), jnp.float32)],
)


def kernel(dense_x, lS_i, emb_tables, bW0, bb0, bW1, bb1, bW2, bb2,
           tW0, tb0, tW1, tb1, tW2, tb2):
    tabT = emb_tables.transpose(0, 2, 1)
    lyT = _sc_gather(tabT, lS_i.astype(jnp.int32))

    xdT = dense_x.T
    tW0p = jnp.pad(tW0, ((0, 0), (0, 1)))
    s8 = jnp.asarray(np.kron(np.eye(8, dtype=np.float32),
                             np.ones((1, D), np.float32)))
    pT = _tc_call(xdT, lyT, s8,
                  bW0, bb0[:, None], bW1, bb1[:, None], bW2, bb2[:, None],
                  tW0p, tb0[:, None], tW1, tb1[:, None],
                  tW2, tb2[:, None])
    return pT.reshape(B, 1)


# speedup vs baseline: 50.0290x; 1.0801x over previous
import functools
import numpy as np
import jax
import jax.numpy as jnp
from jax import lax
from jax.experimental import pallas as pl
from jax.experimental.pallas import tpu as pltpu
from jax.experimental.pallas import tpu_sc as plsc

B = 16384
F = 26
V = 100000
D = 16

NC, NS, L = 2, 16, 16
NW = NC * NS
ROWS = F * D
RPT = ROWS // NW
OCHUNK = 4096

_mesh = plsc.VectorSubcoreMesh(core_axis_name="c", subcore_axis_name="s")


@functools.partial(
    pl.kernel,
    out_type=jax.ShapeDtypeStruct((ROWS, B), jnp.float32),
    mesh=_mesh,
    scratch_types=[
        pltpu.VMEM((V,), jnp.float32),
        pltpu.VMEM((B,), jnp.int32),
        pltpu.VMEM((OCHUNK,), jnp.float32),
        pltpu.VMEM((OCHUNK,), jnp.float32),
        pltpu.SemaphoreType.DMA,
        pltpu.SemaphoreType.DMA,
        pltpu.SemaphoreType.DMA,
        pltpu.SemaphoreType.DMA,
    ],
    compiler_params=pltpu.CompilerParams(needs_layout_passes=False),
)
def _sc_gather(tabT_hbm, idx_hbm, out_hbm, row_v, idx_v, out_v0, out_v1,
               rsem, isem, osem0, osem1):
    wid = lax.axis_index("s") * NC + lax.axis_index("c")
    osems = (osem0, osem1)
    obufs = (out_v0, out_v1)
    pending = [None, None]
    for i in range(RPT):
        r = wid * RPT + i
        f = r // D
        d = r % D
        crow = pltpu.async_copy(tabT_hbm.at[f, d, :], row_v, rsem)
        cidx = pltpu.async_copy(idx_hbm.at[f, :], idx_v, isem)
        crow.wait()
        cidx.wait()
        for h in range(B // OCHUNK):
            ob = obufs[h % 2]
            if pending[h % 2] is not None:
                pending[h % 2].wait()

            @plsc.parallel_loop(0, OCHUNK // L, unroll=4)
            def _gat(j, h=h, ob=ob):
                iv = idx_v[pl.ds(h * OCHUNK + j * L, L)]
                ob[pl.ds(j * L, L)] = plsc.load_gather(row_v, [iv])

            pending[h % 2] = pltpu.async_copy(
                ob, out_hbm.at[r, pl.ds(h * OCHUNK, OCHUNK)], osems[h % 2])
    for h in range(2):
        if pending[h] is not None:
            pending[h].wait()


BB = 1024
NBLK = B // BB
_LI, _LJ = np.tril_indices(F + 1, -1)


def _tc_body(xdT_ref, lyT_ref, s8_ref, bW0_ref, bb0_ref, bW1_ref, bb1_ref,
             bW2_ref, bb2_ref, tW0_ref, tb0_ref, tW1_ref, tb1_ref,
             tW2_ref, tb2_ref, out_ref, rt_ref):
    f32 = jnp.float32
    x = xdT_ref[...]
    h = jnp.maximum(jnp.dot(bW0_ref[...], x, preferred_element_type=f32)
                    + bb0_ref[...], 0.0)
    h = jnp.maximum(jnp.dot(bW1_ref[...], h, preferred_element_type=f32)
                    + bb1_ref[...], 0.0)
    x3 = jnp.maximum(jnp.dot(bW2_ref[...], h, preferred_element_type=f32)
                     + bb2_ref[...], 0.0)

    ly = lyT_ref[...]
    ts = [x3] + [lax.slice(ly, (f * D, 0), ((f + 1) * D, BB))
                 for f in range(F)]

    rt_ref[0:D, :] = x3
    s8 = s8_ref[...]
    npair = len(_LI)
    for g in range((npair + 7) // 8):
        prods = []
        for u in range(8):
            p = g * 8 + u
            i, k = (_LI[p], _LJ[p]) if p < npair else (0, 0)
            prods.append(ts[i] * ts[k])
        zg = jnp.dot(s8, jnp.concatenate(prods, axis=0),
                     preferred_element_type=f32)
        rt_ref[pl.ds(D + g * 8, 8), :] = zg

    r = rt_ref[...]
    h = jnp.maximum(jnp.dot(tW0_ref[...], r, preferred_element_type=f32)
                    + tb0_ref[...], 0.0)
    h = jnp.maximum(jnp.dot(tW1_ref[...], h, preferred_element_type=f32)
                    + tb1_ref[...], 0.0)
    o = jnp.dot(tW2_ref[...], h, preferred_element_type=f32) + tb2_ref[...]
    out_ref[...] = 1.0 / (1.0 + jnp.exp(-o))


def _full(shape):
    return pl.BlockSpec(shape, lambda i: (0,) * len(shape))


_tc_call = pl.pallas_call(
    _tc_body,
    grid=(NBLK,),
    in_specs=[
        pl.BlockSpec((13, BB), lambda i: (0, i)),
        pl.BlockSpec((F * D, BB), lambda i: (0, i)),
        _full((8, 8 * D)),
        _full((512, 13)), _full((512, 1)),
        _full((256, 512)), _full((256, 1)),
        _full((16, 256)), _full((16, 1)),
        _full((512, 368)), _full((512, 1)),
        _full((256, 512)), _full((256, 1)),
        _full((1, 256)), _full((1, 1)),
    ],
    out_specs=pl.BlockSpec((1, BB), lambda i: (0, i)),
    out_shape=jax.ShapeDtypeStruct((1, B), jnp.float32),
    scratch_shapes=[pltpu.VMEM((368, BB), jnp.float32)],
)


def kernel(dense_x, lS_i, emb_tables, bW0, bb0, bW1, bb1, bW2, bb2,
           tW0, tb0, tW1, tb1, tW2, tb2):
    tabT = emb_tables.transpose(0, 2, 1)
    lyT = _sc_gather(tabT, lS_i.astype(jnp.int32))

    xdT = dense_x.T
    tW0p = jnp.pad(tW0, ((0, 0), (0, 1)))
    s8 = jnp.asarray(np.kron(np.eye(8, dtype=np.float32),
                             np.ones((1, D), np.float32)))
    pT = _tc_call(xdT, lyT, s8,
                  bW0, bb0[:, None], bW1, bb1[:, None], bW2, bb2[:, None],
                  tW0p, tb0[:, None], tW1, tb1[:, None],
                  tW2, tb2[:, None])
    return pT.reshape(B, 1)
